# baseline (speedup 1.0000x reference)
import jax
import jax.numpy as jnp
from jax.experimental import pallas as pl
from jax.experimental.pallas import tpu as pltpu


def _pick_block(n):
    for b in (400, 200, 100, 8, 4, 2, 1):
        if n % b == 0:
            return b
    return n


def _gcn_kernel(x_hbm, adj_hbm, w_ref, b_ref, out_hbm,
                xbuf, xw_ref, abuf, ostage, asems, xsem, osems):
    n = adj_hbm.shape[0]
    bm = abuf.shape[1]
    nchunks = n // bm

    def _acopy(chunk, slot):
        return pltpu.make_async_copy(
            adj_hbm.at[pl.ds(chunk * bm, bm), :], abuf.at[slot], asems.at[slot]
        )

    def _ocopy(chunk, slot):
        return pltpu.make_async_copy(
            ostage.at[slot], out_hbm.at[pl.ds(chunk * bm, bm), :], osems.at[slot]
        )

    _acopy(0, 0).start()
    _acopy(1, 1).start()
    xcopy = pltpu.make_async_copy(x_hbm, xbuf, xsem)
    xcopy.start()
    xcopy.wait()
    xw_ref[...] = jax.lax.dot(
        xbuf[...], w_ref[...], preferred_element_type=jnp.float32
    )

    def step(i, carry):
        slot = jax.lax.rem(i, 2)
        _acopy(i, slot).wait()
        acc = jax.lax.dot(abuf[slot], xw_ref[...],
                          preferred_element_type=jnp.float32)

        @pl.when(i + 2 < nchunks)
        def _prefetch():
            _acopy(i + 2, slot).start()

        @pl.when(i >= 2)
        def _drain():
            _ocopy(i - 2, slot).wait()

        ostage[slot] = acc + b_ref[...]
        _ocopy(i, slot).start()
        return carry

    jax.lax.fori_loop(0, nchunks, step, 0)
    _ocopy(nchunks - 2, jax.lax.rem(nchunks - 2, 2)).wait()
    _ocopy(nchunks - 1, jax.lax.rem(nchunks - 1, 2)).wait()


def kernel(input, adj, weight, bias):
    n, f_in = input.shape
    f_out = weight.shape[1]
    bm = _pick_block(n)
    bias2 = bias.reshape(1, f_out)
    return pl.pallas_call(
        _gcn_kernel,
        in_specs=[
            pl.BlockSpec(memory_space=pl.ANY),
            pl.BlockSpec(memory_space=pl.ANY),
            pl.BlockSpec(memory_space=pltpu.VMEM),
            pl.BlockSpec(memory_space=pltpu.VMEM),
        ],
        out_specs=pl.BlockSpec(memory_space=pl.ANY),
        out_shape=jax.ShapeDtypeStruct((n, f_out), jnp.float32),
        compiler_params=pltpu.CompilerParams(
            vmem_limit_bytes=64 * 1024 * 1024,
        ),
        scratch_shapes=[
            pltpu.VMEM((n, f_in), jnp.float32),
            pltpu.VMEM((n, f_out), jnp.float32),
            pltpu.VMEM((2, bm, n), jnp.float32),
            pltpu.VMEM((2, bm, f_out), jnp.float32),
            pltpu.SemaphoreType.DMA((2,)),
            pltpu.SemaphoreType.DMA,
            pltpu.SemaphoreType.DMA((2,)),
        ],
    )(input, adj, weight, bias2)


# speedup vs baseline: 1.0493x; 1.0493x over previous
import jax
import jax.numpy as jnp
from jax.experimental import pallas as pl
from jax.experimental.pallas import tpu as pltpu


def _gcn_kernel(x_ref, adj_ref, w_ref, b_ref, out_ref, xw_ref):
    i = pl.program_id(0)

    @pl.when(i == 0)
    def _compute_xw():
        xw_ref[...] = jax.lax.dot(
            x_ref[...], w_ref[...], preferred_element_type=jnp.float32
        )

    acc = jax.lax.dot(adj_ref[...], xw_ref[...],
                      preferred_element_type=jnp.float32)
    out_ref[...] = acc + b_ref[...]


def _pick_block(n):
    for b in (400, 200, 100, 8, 4, 2, 1):
        if n % b == 0:
            return b
    return n


def kernel(input, adj, weight, bias):
    n, f_in = input.shape
    f_out = weight.shape[1]
    bm = _pick_block(n)
    bias2 = bias.reshape(1, f_out)
    grid = (n // bm,)
    return pl.pallas_call(
        _gcn_kernel,
        grid=grid,
        in_specs=[
            pl.BlockSpec((n, f_in), lambda i: (0, 0)),
            pl.BlockSpec((bm, n), lambda i: (i, 0)),
            pl.BlockSpec((f_in, f_out), lambda i: (0, 0)),
            pl.BlockSpec((1, f_out), lambda i: (0, 0)),
        ],
        out_specs=pl.BlockSpec((bm, f_out), lambda i: (i, 0)),
        out_shape=jax.ShapeDtypeStruct((n, f_out), jnp.float32),
        scratch_shapes=[pltpu.VMEM((n, f_out), jnp.float32)],
    )(input, adj, weight, bias2)
